# baseline (speedup 1.0000x reference)
import functools

import jax
import jax.numpy as jnp
from jax.experimental import pallas as pl
from jax.experimental.pallas import tpu as pltpu

K = 16


def _phase1_body(nb, assign_ref, vec_ref, cent_ref, sums_acc, counts_acc):
    i = pl.program_id(0)

    @pl.when(i == 0)
    def _init():
        sums_acc[...] = jnp.zeros_like(sums_acc)
        counts_acc[...] = jnp.zeros_like(counts_acc)

    a = assign_ref[0]
    kio = jax.lax.broadcasted_iota(jnp.int32, (K, a.shape[1]), 0)
    onehot = (a == kio).astype(jnp.bfloat16)

    v = vec_ref[...]
    vhi = v.astype(jnp.bfloat16)
    r1 = v - vhi.astype(jnp.float32)
    vmid = r1.astype(jnp.bfloat16)
    vlo = (r1 - vmid.astype(jnp.float32)).astype(jnp.bfloat16)

    dn = (((1,), (0,)), ((), ()))
    part = (
        jax.lax.dot_general(onehot, vhi, dn, preferred_element_type=jnp.float32)
        + jax.lax.dot_general(onehot, vmid, dn, preferred_element_type=jnp.float32)
        + jax.lax.dot_general(onehot, vlo, dn, preferred_element_type=jnp.float32)
    )
    sums_acc[...] += part
    counts_acc[...] += jnp.sum(
        onehot.astype(jnp.float32), axis=1, keepdims=True)

    @pl.when(i == nb - 1)
    def _fin():
        cent_ref[...] = sums_acc[...] / counts_acc[...]


def _phase2_body(cent_ref, vec_ref, out_ref):
    c = cent_ref[...].astype(jnp.bfloat16)
    cf = cent_ref[...]
    c2 = jnp.sum(cf * cf, axis=1, keepdims=True)
    cross = jax.lax.dot_general(
        c, vec_ref[...].astype(jnp.bfloat16), (((1,), (1,)), ((), ())),
        preferred_element_type=jnp.float32)
    score = c2 - 2.0 * cross
    min_v = jnp.min(score, axis=0, keepdims=True)
    kio = jax.lax.broadcasted_iota(jnp.int32, score.shape, 0)
    idx = jnp.min(jnp.where(score == min_v, kio, K), axis=0, keepdims=True)
    out_ref[...] = idx[None]


def kernel(vectors, assignment):
    N, D = vectors.shape
    B = 16384
    nb = N // B
    assign3 = assignment.reshape(nb, 1, B)

    centroids = pl.pallas_call(
        functools.partial(_phase1_body, nb),
        grid=(nb,),
        in_specs=[
            pl.BlockSpec((1, 1, B), lambda i: (i, 0, 0)),
            pl.BlockSpec((B, D), lambda i: (i, 0)),
        ],
        out_specs=pl.BlockSpec((K, D), lambda i: (0, 0)),
        out_shape=jax.ShapeDtypeStruct((K, D), jnp.float32),
        scratch_shapes=[
            pltpu.VMEM((K, D), jnp.float32),
            pltpu.VMEM((K, 1), jnp.float32),
        ],
    )(assign3, vectors)

    new_assign3 = pl.pallas_call(
        _phase2_body,
        grid=(nb,),
        in_specs=[
            pl.BlockSpec((K, D), lambda i: (0, 0)),
            pl.BlockSpec((B, D), lambda i: (i, 0)),
        ],
        out_specs=pl.BlockSpec((1, 1, B), lambda i: (i, 0, 0)),
        out_shape=jax.ShapeDtypeStruct((nb, 1, B), jnp.int32),
    )(centroids, vectors)

    return centroids, new_assign3.reshape(N)
